# baseline (speedup 1.0000x reference)
import functools

import jax
import jax.numpy as jnp
from jax import lax
from jax.experimental import pallas as pl
from jax.experimental.pallas import tpu as pltpu
from jax.experimental.pallas import tpu_sc as plsc

N_NODES = 100000
D_IN = 128
HID = 256
NCLS = 47
N0 = 495616
N1 = 45056
N2 = 4096
E0 = 450560
E1 = 40960

NC = 2
NS = 16
NW = NC * NS

E0_PT = E0 // NW
E0_G = E0_PT // 128
XT_PT = N1 // NW
XT_G = XT_PT // 128
E1_PT = E1 // NW
E1_G = E1_PT // 128


def _scgat1_body(gid0_hbm, esrc_hbm, table_hbm,
                 msg_hbm, xt_hbm,
                 b2d, gid1d, rows2, semg):
    cid = lax.axis_index("c")
    sid = lax.axis_index("s")
    wid = cid * NS + sid

    pltpu.sync_copy(esrc_hbm.at[wid], b2d)
    def _gidg(j10, _):
        cps = []
        for k in range(10):
            j = j10 * 10 + k
            cps.append(pltpu.async_copy(
                gid0_hbm.at[b2d.at[j]],
                gid1d.at[pl.ds(j * 128, 128)], semg))
        for cp in cps:
            cp.wait()
        return 0
    lax.fori_loop(0, E0_G // 10, _gidg, 0)

    def _mg(j2, _):
        ja = j2 * 2
        jb = ja + 1
        ga = pltpu.async_copy(
            table_hbm.at[gid1d.at[pl.ds(ja * 128, 128)]], rows2.at[0], semg)
        ga.wait()
        wa = pltpu.async_copy(
            rows2.at[0],
            msg_hbm.at[pl.ds(wid * E0_PT + ja * 128, 128)], semg)
        gb = pltpu.async_copy(
            table_hbm.at[gid1d.at[pl.ds(jb * 128, 128)]], rows2.at[1], semg)
        gb.wait()
        wb = pltpu.async_copy(
            rows2.at[1],
            msg_hbm.at[pl.ds(wid * E0_PT + jb * 128, 128)], semg)
        wa.wait()
        wb.wait()
        return 0
    lax.fori_loop(0, E0_G // 2, _mg, 0)

    pltpu.sync_copy(gid0_hbm.at[pl.ds(wid * XT_PT, XT_PT)],
                    gid1d.at[pl.ds(0, XT_PT)])
    def _xtg(j2, _):
        ja = j2 * 2
        jb = ja + 1
        ga = pltpu.async_copy(
            table_hbm.at[gid1d.at[pl.ds(ja * 128, 128)]], rows2.at[0], semg)
        ga.wait()
        wa = pltpu.async_copy(
            rows2.at[0], xt_hbm.at[pl.ds(wid * XT_PT + ja * 128, 128)], semg)
        gb = pltpu.async_copy(
            table_hbm.at[gid1d.at[pl.ds(jb * 128, 128)]], rows2.at[1], semg)
        gb.wait()
        wb = pltpu.async_copy(
            rows2.at[1], xt_hbm.at[pl.ds(wid * XT_PT + jb * 128, 128)], semg)
        wa.wait()
        wb.wait()
        return 0
    lax.fori_loop(0, XT_G // 2, _xtg, 0)
    ga = pltpu.async_copy(
        table_hbm.at[gid1d.at[pl.ds((XT_G - 1) * 128, 128)]],
        rows2.at[0], semg)
    ga.wait()
    pltpu.sync_copy(rows2.at[0],
                    xt_hbm.at[pl.ds(wid * XT_PT + (XT_G - 1) * 128, 128)])


_scgat1 = functools.partial(
    pl.kernel,
    out_type=[jax.ShapeDtypeStruct((E0, D_IN), jnp.float32),
              jax.ShapeDtypeStruct((N1, D_IN), jnp.float32)],
    mesh=plsc.VectorSubcoreMesh(core_axis_name="c", subcore_axis_name="s"),
    scratch_types=[
        pltpu.VMEM((E0_G, 128), jnp.int32),
        pltpu.VMEM((E0_PT,), jnp.int32),
        pltpu.VMEM((2, 128, 128), jnp.float32),
        pltpu.SemaphoreType.DMA,
    ],
)(_scgat1_body)


def _scgat2_body(h_hbm, esrc_hbm, msg_hbm, s2d, rows2, semg):
    cid = lax.axis_index("c")
    sid = lax.axis_index("s")
    wid = cid * NS + sid

    pltpu.sync_copy(esrc_hbm.at[wid], s2d)

    def _mg(j2, _):
        ja = j2 * 2
        jb = ja + 1
        ga = pltpu.async_copy(h_hbm.at[s2d.at[ja]], rows2.at[0], semg)
        ga.wait()
        wa = pltpu.async_copy(
            rows2.at[0],
            msg_hbm.at[pl.ds(wid * E1_PT + ja * 128, 128)], semg)
        gb = pltpu.async_copy(h_hbm.at[s2d.at[jb]], rows2.at[1], semg)
        gb.wait()
        wb = pltpu.async_copy(
            rows2.at[1],
            msg_hbm.at[pl.ds(wid * E1_PT + jb * 128, 128)], semg)
        wa.wait()
        wb.wait()
        return 0
    lax.fori_loop(0, E1_G // 2, _mg, 0)


_scgat2 = functools.partial(
    pl.kernel,
    out_type=[jax.ShapeDtypeStruct((E1, HID), jnp.float32)],
    mesh=plsc.VectorSubcoreMesh(core_axis_name="c", subcore_axis_name="s"),
    scratch_types=[
        pltpu.VMEM((E1_G, 128), jnp.int32),
        pltpu.VMEM((2, 128, HID), jnp.float32),
        pltpu.SemaphoreType.DMA,
    ],
)(_scgat2_body)


_BLK1 = 512


def _tc1_body(agg_ref, deg_ref, xt_ref, wn_ref, ws_ref, b_ref, o_ref):
    rdeg = 1.0 / jnp.maximum(deg_ref[...], 1.0)
    a = agg_ref[...] * rdeg
    h = jnp.dot(a, wn_ref[...], preferred_element_type=jnp.float32)
    h = h + jnp.dot(xt_ref[...], ws_ref[...], preferred_element_type=jnp.float32)
    h = h + b_ref[...]
    o_ref[...] = jnp.maximum(h, 0.0)


def _tc1(agg, deg, xt, wn, ws, b):
    return pl.pallas_call(
        _tc1_body,
        grid=(N1 // _BLK1,),
        in_specs=[
            pl.BlockSpec((_BLK1, D_IN), lambda i: (i, 0)),
            pl.BlockSpec((_BLK1, 1), lambda i: (i, 0)),
            pl.BlockSpec((_BLK1, D_IN), lambda i: (i, 0)),
            pl.BlockSpec((D_IN, HID), lambda i: (0, 0)),
            pl.BlockSpec((D_IN, HID), lambda i: (0, 0)),
            pl.BlockSpec((1, HID), lambda i: (0, 0)),
        ],
        out_specs=pl.BlockSpec((_BLK1, HID), lambda i: (i, 0)),
        out_shape=jax.ShapeDtypeStruct((N1, HID), jnp.float32),
    )(agg, deg, xt, wn, ws, b)


def _tc2_body(agg_ref, deg_ref, h_ref, wn_ref, ws_ref, b_ref, o_ref):
    rdeg = 1.0 / jnp.maximum(deg_ref[...], 1.0)
    a = agg_ref[...] * rdeg
    o = jnp.dot(a, wn_ref[...], preferred_element_type=jnp.float32)
    o = o + jnp.dot(h_ref[...], ws_ref[...], preferred_element_type=jnp.float32)
    o_ref[...] = o + b_ref[...]


def _tc2(agg, deg, h, wn, ws, b):
    return pl.pallas_call(
        _tc2_body,
        grid=(N2 // _BLK1,),
        in_specs=[
            pl.BlockSpec((_BLK1, HID), lambda i: (i, 0)),
            pl.BlockSpec((_BLK1, 1), lambda i: (i, 0)),
            pl.BlockSpec((_BLK1, HID), lambda i: (i, 0)),
            pl.BlockSpec((HID, 128), lambda i: (0, 0)),
            pl.BlockSpec((HID, 128), lambda i: (0, 0)),
            pl.BlockSpec((1, 128), lambda i: (0, 0)),
        ],
        out_specs=pl.BlockSpec((_BLK1, 128), lambda i: (i, 0)),
        out_shape=jax.ShapeDtypeStruct((N2, 128), jnp.float32),
    )(agg, deg, h, wn, ws, b)


def kernel(target_gid0, edge_src0, edge_dst0, edge_src1, edge_dst1, table,
           W_neigh0, W_self0, b0, W_neigh1, W_self1, b1):
    gid0 = target_gid0.astype(jnp.int32)
    src0 = edge_src0.astype(jnp.int32)
    dst0 = edge_dst0.astype(jnp.int32)
    src1 = edge_src1.astype(jnp.int32)
    dst1 = edge_dst1.astype(jnp.int32)

    msg0, xt = _scgat1(gid0, src0.reshape(NW, E0_G, 128), table)
    agg0 = jax.ops.segment_sum(msg0, dst0, num_segments=N1)
    deg0 = jax.ops.segment_sum(jnp.ones((E0,), jnp.float32), dst0,
                               num_segments=N1)[:, None]
    h = _tc1(agg0, deg0, xt, W_neigh0, W_self0, b0.reshape(1, HID))

    msg1, = _scgat2(h, src1.reshape(NW, E1_G, 128))
    agg1 = jax.ops.segment_sum(msg1, dst1, num_segments=N2)
    deg1 = jax.ops.segment_sum(jnp.ones((E1,), jnp.float32), dst1,
                               num_segments=N2)[:, None]
    wn1 = jnp.pad(W_neigh1, ((0, 0), (0, 128 - NCLS)))
    ws1 = jnp.pad(W_self1, ((0, 0), (0, 128 - NCLS)))
    b1p = jnp.pad(b1, (0, 128 - NCLS)).reshape(1, 128)
    out = _tc2(agg1, deg1, h, wn1, ws1, b1p)
    return out[:, :NCLS]
